# Initial kernel scaffold; baseline (speedup 1.0000x reference)
import jax
import jax.numpy as jnp
from jax.experimental import pallas as pl


def kernel(x, edge_index, conv1, conv4, mlp_params, heads_params):
    raise NotImplementedError("write your pallas kernel here")


# speedup vs baseline: 46.1404x; 46.1404x over previous
import functools

import jax
import jax.numpy as jnp
from jax import lax
from jax.experimental import pallas as pl
from jax.experimental.pallas import tpu as pltpu
from jax.experimental.pallas import tpu_sc as plsc

_NC = 2
_NS = 16
_NW = _NC * _NS


def _tc_pre(x, W, A_src, A_dst):
    n = x.shape[0]
    HC = W.shape[1]

    F = x.shape[1]
    BM = 1000

    def body(x_ref, w_ref, as_ref, ad_ref, h_ref, asrc_ref, adst_ref, pself_ref):
        h = jnp.dot(x_ref[:].astype(jnp.bfloat16), w_ref[:].astype(jnp.bfloat16),
                    preferred_element_type=jnp.float32)
        h_ref[:] = h
        asrc = jnp.dot(h, as_ref[:], preferred_element_type=jnp.float32, precision=lax.Precision.HIGHEST)
        adst = jnp.dot(h, ad_ref[:], preferred_element_type=jnp.float32, precision=lax.Precision.HIGHEST)
        asrc_ref[:] = asrc
        adst_ref[:] = adst
        t = asrc + adst
        t = jnp.where(t >= 0, t, 0.2 * t)
        pself_ref[:] = jnp.exp(t)

    return pl.pallas_call(
        body,
        grid=(n // BM,),
        in_specs=[
            pl.BlockSpec((BM, F), lambda i: (i, 0)),
            pl.BlockSpec((F, HC), lambda i: (0, 0)),
            pl.BlockSpec((HC, 16), lambda i: (0, 0)),
            pl.BlockSpec((HC, 16), lambda i: (0, 0)),
        ],
        out_specs=[
            pl.BlockSpec((BM, HC), lambda i: (i, 0)),
            pl.BlockSpec((BM, 16), lambda i: (i, 0)),
            pl.BlockSpec((BM, 16), lambda i: (i, 0)),
            pl.BlockSpec((BM, 16), lambda i: (i, 0)),
        ],
        out_shape=[
            jax.ShapeDtypeStruct((n, HC), jnp.float32),
            jax.ShapeDtypeStruct((n, 16), jnp.float32),
            jax.ShapeDtypeStruct((n, 16), jnp.float32),
            jax.ShapeDtypeStruct((n, 16), jnp.float32),
        ],
    )(x, W, A_src, A_dst)


def _tc_post(msgp, sp, pself, h, bias, Ex):
    n, HC = h.shape

    BM = 1000

    def body(msg0_ref, msg1_ref, s0_ref, s1_ref, pself_ref, h_ref, b_ref,
             ex_ref, out_ref):
        ex = ex_ref[:]
        pself_v = pself_ref[:]
        pex = jnp.dot(pself_v, ex, preferred_element_type=jnp.float32, precision=lax.Precision.HIGHEST)
        msg = msg0_ref[:] + msg1_ref[:] + pex * h_ref[:]
        s = s0_ref[:] + s1_ref[:] + pself_v
        sex = jnp.dot(s, ex, preferred_element_type=jnp.float32, precision=lax.Precision.HIGHEST)
        o = msg / (sex + 1e-16) + b_ref[:]
        out_ref[:] = jnp.where(o > 0, o, jnp.exp(jnp.minimum(o, 0.0)) - 1.0)

    wide = pl.BlockSpec((BM, HC), lambda i: (i, 0))
    narrow = pl.BlockSpec((BM, 16), lambda i: (i, 0))
    return pl.pallas_call(
        body,
        grid=(n // BM,),
        in_specs=[wide, wide, narrow, narrow, narrow, wide,
                  pl.BlockSpec((1, HC), lambda i: (0, 0)),
                  pl.BlockSpec((16, HC), lambda i: (0, 0))],
        out_specs=wide,
        out_shape=jax.ShapeDtypeStruct((n, HC), jnp.float32),
    )(msgp[0], msgp[1], sp[0], sp[1], pself, h, bias, Ex)


def _sc_edges(src, dst, asrc, adst, h, HC, C):
    E = src.shape[0]
    n = h.shape[0]
    K = 80
    EPW = E // _NW
    NCH = EPW // K
    NV = HC // 16

    mesh = plsc.VectorSubcoreMesh(core_axis_name="c", subcore_axis_name="s")

    @functools.partial(
        pl.kernel,
        mesh=mesh,
        compiler_params=pltpu.CompilerParams(use_tc_tiling_on_sc=False),
        out_type=[
            jax.ShapeDtypeStruct((_NC, n, HC), jnp.float32),
            jax.ShapeDtypeStruct((_NC, n, 16), jnp.float32),
        ],
        scratch_types=[
            pltpu.VMEM((NV, 16), jnp.int32),
            pltpu.VMEM((K,), jnp.int32),
            pltpu.VMEM((K,), jnp.int32),
            pltpu.VMEM((K, 16), jnp.float32),
            pltpu.VMEM((K, 16), jnp.float32),
            pltpu.VMEM((K, 16), jnp.float32),
            pltpu.VMEM((K, HC), jnp.float32),
            pltpu.VMEM((K, HC), jnp.float32),
            pltpu.VMEM_SHARED((n, HC), jnp.float32),
            pltpu.VMEM_SHARED((n, 16), jnp.float32),
            pltpu.SemaphoreType.DMA,
            pltpu.SemaphoreType.DMA,
            pltpu.SemaphoreType.DMA,
        ],
    )
    def ek(src_hbm, dst_hbm, asrc_hbm, adst_hbm, h_hbm, hidx_hbm, zmsg_hbm,
           zs_hbm, msg_out, s_out,
           hidx_v, src_v, dst_v, asrc_v, adst_v, p_v, h_v, msg_v,
           msg_sh, s_sh, sem_a, sem_b, sem_h):
        cid = lax.axis_index("c")
        sid = lax.axis_index("s")
        wid = sid * _NC + cid

        pltpu.sync_copy(hidx_hbm, hidx_v)

        @pl.when(sid == 0)
        def _zero():
            pltpu.sync_copy(zmsg_hbm, msg_sh)
            pltpu.sync_copy(zs_hbm, s_sh)

        plsc.subcore_barrier()

        def chunk(t, carry):
            base = wid * EPW + t * K
            pltpu.sync_copy(src_hbm.at[pl.ds(base, K)], src_v)
            pltpu.sync_copy(dst_hbm.at[pl.ds(base, K)], dst_v)
            ca = pltpu.async_copy(asrc_hbm.at[src_v], asrc_v, sem_a)
            cb = pltpu.async_copy(adst_hbm.at[dst_v], adst_v, sem_b)
            ch = pltpu.async_copy(h_hbm.at[src_v], h_v, sem_h)
            ca.wait()
            cb.wait()
            ch.wait()

            def body(j, c2):
                a = asrc_v[j, :] + adst_v[j, :]
                a = jnp.where(a >= 0, a, 0.2 * a)
                p = jnp.exp(a)
                p_v[j, :] = p
                for v in range(NV):
                    alpha = p.at[hidx_v[v, :]].get(mode="promise_in_bounds")
                    msg_v[j, pl.ds(16 * v, 16)] = h_v[j, pl.ds(16 * v, 16)] * alpha
                return c2

            lax.fori_loop(0, K, body, 0)
            pltpu.sync_copy(p_v, s_sh.at[dst_v], add=True)
            pltpu.sync_copy(msg_v, msg_sh.at[dst_v], add=True)
            return carry

        lax.fori_loop(0, NCH, chunk, 0)
        plsc.subcore_barrier()

        @pl.when(sid == 0)
        def _flush():
            pltpu.sync_copy(msg_sh, msg_out.at[cid])
            pltpu.sync_copy(s_sh, s_out.at[cid])

    hidx_tab = jnp.array(
        [[(16 * v + l) // C for l in range(16)] for v in range(NV)], jnp.int32)
    zmsg = jnp.zeros((n, HC), jnp.float32)
    zs = jnp.zeros((n, 16), jnp.float32)
    return ek(src, dst, asrc, adst, h, hidx_tab, zmsg, zs)


def _gat_layer(x, src, dst, conv, H, C):
    W, a_src, a_dst, bias = conv
    HC = H * C
    onehot = jnp.eye(H, 16, dtype=jnp.float32)
    A_src = (a_src[:, :, None] * onehot[:, None, :]).reshape(HC, 16)
    A_dst = (a_dst[:, :, None] * onehot[:, None, :]).reshape(HC, 16)
    Ex = jnp.repeat(jnp.eye(16, H, dtype=jnp.float32), C, axis=1)
    h, asrc, adst, pself = _tc_pre(x, W, A_src, A_dst)
    msgp, sp = _sc_edges(src, dst, asrc, adst, h, HC, C)
    return _tc_post(msgp, sp, pself, h, bias.reshape(1, HC), Ex)


def _mlp_heads(x, mlp_params, heads_params):
    n = x.shape[0]
    flat = []
    for (W, b, g, bt) in mlp_params:
        flat += [W, b.reshape(1, -1), g.reshape(1, -1), bt.reshape(1, -1)]
    for head in heads_params:
        for (W, b, g, bt) in head:
            flat += [W, b.reshape(1, -1), g.reshape(1, -1), bt.reshape(1, -1)]

    n_mlp = len(mlp_params)
    n_head_layers = len(heads_params[0])
    n_heads = len(heads_params)

    def chain(v, params):
        for (Wv, bv, gv, btv) in params:
            y = jnp.maximum(
                jnp.dot(v.astype(jnp.bfloat16), Wv.astype(jnp.bfloat16),
                        preferred_element_type=jnp.float32) + bv, 0.0)
            mu = jnp.mean(y, axis=0, keepdims=True)
            var = jnp.mean((y - mu) ** 2, axis=0, keepdims=True)
            v = gv * (y - mu) * lax.rsqrt(var + 1e-5) + btv
        return v

    def body(x_ref, *refs):
        out_ref = refs[-1]
        vals = [r[:] for r in refs[:-1]]
        mlp = [tuple(vals[4 * i:4 * i + 4]) for i in range(n_mlp)]
        off = 4 * n_mlp
        heads = []
        for hh in range(n_heads):
            layers = []
            for i in range(n_head_layers):
                base = off + 4 * (hh * n_head_layers + i)
                layers.append(tuple(vals[base:base + 4]))
            heads.append(layers)
        v = chain(x_ref[:], mlp)
        outs = [chain(v, head) for head in heads]
        out_ref[:] = jnp.concatenate(outs, axis=1)

    return pl.pallas_call(
        body,
        out_shape=jax.ShapeDtypeStruct((n, n_heads), jnp.float32),
    )(x, *flat)


def kernel(x, edge_index, conv1, conv4, mlp_params, heads_params):
    src = edge_index[0]
    dst = edge_index[1]
    h1 = _gat_layer(x, src, dst, conv1, 12, 12)
    h2 = _gat_layer(h1, src, dst, conv4, 8, 8)
    return _mlp_heads(h2, mlp_params, heads_params)
